# baseline (speedup 1.0000x reference)
import dataclasses

import jax
import jax.numpy as jnp
from jax import lax
from jax.experimental import pallas as pl
from jax.experimental.pallas import tpu as pltpu
from jax.experimental.pallas import tpu_sc as plsc

D = 128
K = 20
NC = 2
NS = 16
NW = NC * NS
L = 16
CHUNK = 16
NSPLIT = (112, 112, 96)


def _build_sc_call(B):
    spw = B // NW
    n_chunks = spw // CHUNK
    n_rows = CHUNK * K

    mesh = plsc.VectorSubcoreMesh(core_axis_name="c", subcore_axis_name="s")

    def body(cw_hbm, ctx_hbm, neg_hbm, cemb_hbm, uemb_hbm,
             pos_out, neg_out,
             cidx_v, xidx_v, nidx_v, cbuf, xbuf, nbuf,
             stage_p, stage_n, opos, oneg, sem0, sem1):
        wid = lax.axis_index("s") * NC + lax.axis_index("c")
        sbase = wid * spw
        pltpu.sync_copy(cw_hbm.at[pl.ds(sbase, spw)], cidx_v)
        pltpu.sync_copy(ctx_hbm.at[pl.ds(sbase, spw)], xidx_v)
        pltpu.sync_copy(neg_hbm.at[pl.ds(sbase * K, spw * K)], nidx_v)

        lanes = lax.iota(jnp.int32, L)
        scat_p = lanes * CHUNK
        scat_n = lanes * n_rows
        sems = (sem0, sem1)

        def fire(g, b):
            pltpu.async_copy(
                cemb_hbm.at[cidx_v.at[pl.ds(g * CHUNK, CHUNK)]],
                cbuf.at[b], sems[b])
            pltpu.async_copy(
                uemb_hbm.at[xidx_v.at[pl.ds(g * CHUNK, CHUNK)]],
                xbuf.at[b], sems[b])
            off = 0
            for w in NSPLIT:
                pltpu.async_copy(
                    uemb_hbm.at[nidx_v.at[pl.ds(g * n_rows + off, w)]],
                    nbuf.at[b, pl.ds(off, w)], sems[b])
                off += w

        def drain(b):
            pltpu.make_async_copy(
                cemb_hbm.at[pl.ds(0, CHUNK)], cbuf.at[b], sems[b]).wait()
            pltpu.make_async_copy(
                uemb_hbm.at[pl.ds(0, CHUNK)], xbuf.at[b], sems[b]).wait()
            pltpu.make_async_copy(
                uemb_hbm.at[pl.ds(0, n_rows)], nbuf.at[b], sems[b]).wait()

        def compute(g, b):
            @plsc.parallel_loop(0, CHUNK, unroll=2)
            def _sample(i):
                v = [cbuf[b, i, pl.ds(t * L, L)] for t in range(D // L)]

                def dot(ref, row):
                    p = [v[t] * ref[b, row, pl.ds(t * L, L)]
                         for t in range(D // L)]
                    while len(p) > 1:
                        p = [p[t] + p[t + 1] for t in range(0, len(p), 2)]
                    return p[0]

                plsc.store_scatter(stage_p, [scat_p + i], dot(xbuf, i))

                @plsc.parallel_loop(0, K, unroll=4)
                def _negs(k):
                    row = i * K + k
                    plsc.store_scatter(stage_n, [scat_n + row],
                                       dot(nbuf, row))

            s = stage_p[pl.ds(0, L)]
            for t in range(1, L):
                s = s + stage_p[pl.ds(t * CHUNK, L)]
            s = jnp.minimum(jnp.maximum(s, -10.0), 10.0)
            opos[pl.ds(g * CHUNK, L)] = s

            @plsc.parallel_loop(0, n_rows // L, unroll=2)
            def _reduce(q):
                s = stage_n[pl.ds(q * L, L)]
                for t in range(1, L):
                    s = s + stage_n[pl.ds(t * n_rows + q * L, L)]
                s = jnp.minimum(jnp.maximum(s, -10.0), 10.0)
                oneg[pl.ds(g * n_rows + q * L, L)] = s

        fire(0, 0)

        @pl.loop(0, n_chunks, step=2)
        def _pair(g):
            fire(g + 1, 1)
            drain(0)
            compute(g, 0)

            @pl.when(g + 2 < n_chunks)
            def _():
                fire(g + 2, 0)

            drain(1)
            compute(g + 1, 1)

        pltpu.sync_copy(opos, pos_out.at[pl.ds(sbase, spw)])
        pltpu.sync_copy(oneg, neg_out.at[pl.ds(sbase * K, spw * K)])

    cp = pltpu.CompilerParams()
    if "needs_layout_passes" in pltpu.CompilerParams.__dataclass_fields__:
        cp = dataclasses.replace(cp, needs_layout_passes=False)
    return pl.kernel(
        body,
        out_type=(jax.ShapeDtypeStruct((B,), jnp.float32),
                  jax.ShapeDtypeStruct((B * K,), jnp.float32)),
        mesh=mesh,
        compiler_params=cp,
        scratch_types=[
            pltpu.VMEM((spw,), jnp.int32),
            pltpu.VMEM((spw,), jnp.int32),
            pltpu.VMEM((spw * K,), jnp.int32),
            pltpu.VMEM((2, CHUNK, D), jnp.float32),
            pltpu.VMEM((2, CHUNK, D), jnp.float32),
            pltpu.VMEM((2, n_rows, D), jnp.float32),
            pltpu.VMEM((L * CHUNK,), jnp.float32),
            pltpu.VMEM((L * n_rows,), jnp.float32),
            pltpu.VMEM((spw,), jnp.float32),
            pltpu.VMEM((spw * K,), jnp.float32),
            pltpu.SemaphoreType.DMA,
            pltpu.SemaphoreType.DMA,
        ],
    )


def kernel(center_words, context_words, negative_samples, center_emb,
           context_emb):
    B = center_words.shape[0]
    cw = center_words.astype(jnp.int32)
    cx = context_words.astype(jnp.int32)
    ng = negative_samples.astype(jnp.int32).reshape(B * K)
    pos, neg = _build_sc_call(B)(cw, cx, ng, center_emb, context_emb)
    return pos, neg.reshape(B, K)


# speedup vs baseline: 1.2080x; 1.2080x over previous
import dataclasses

import jax
import jax.numpy as jnp
from jax import lax
from jax.experimental import pallas as pl
from jax.experimental.pallas import tpu as pltpu
from jax.experimental.pallas import tpu_sc as plsc

D = 128
K = 20
R = K + 1
NC = 2
NS = 16
NW = NC * NS
L = 16
CHUNK = 16
GSPLIT = 112


def _build_sc_call(B):
    spw = B // NW
    n_chunks = spw // CHUNK
    u_rows = CHUNK * R

    mesh = plsc.VectorSubcoreMesh(core_axis_name="c", subcore_axis_name="s")

    def body(cw_hbm, uidx_hbm, cemb_hbm, uemb_hbm, pos_out, neg_out,
             cidx_v, uidx_v, cbuf, ubuf, stage, opos, oneg, sem0, sem1):
        wid = lax.axis_index("s") * NC + lax.axis_index("c")
        sbase = wid * spw
        pltpu.sync_copy(cw_hbm.at[pl.ds(sbase, spw)], cidx_v)
        pltpu.sync_copy(uidx_hbm.at[pl.ds(sbase * R, spw * R)], uidx_v)

        lanes = lax.iota(jnp.int32, L)
        scat_base = lanes * u_rows
        scat_o = lanes * K
        sems = (sem0, sem1)

        def fire(g, b):
            pltpu.async_copy(
                cemb_hbm.at[cidx_v.at[pl.ds(g * CHUNK, CHUNK)]],
                cbuf.at[b], sems[b])
            for p in range(u_rows // GSPLIT):
                pltpu.async_copy(
                    uemb_hbm.at[uidx_v.at[pl.ds(g * u_rows + p * GSPLIT,
                                                GSPLIT)]],
                    ubuf.at[b, pl.ds(p * GSPLIT, GSPLIT)], sems[b])

        def drain(b):
            pltpu.make_async_copy(
                cemb_hbm.at[pl.ds(0, CHUNK)], cbuf.at[b], sems[b]).wait()
            pltpu.make_async_copy(
                uemb_hbm.at[pl.ds(0, u_rows)], ubuf.at[b], sems[b]).wait()

        def compute(g, b):
            @plsc.parallel_loop(0, CHUNK, unroll=2)
            def _sample(i):
                v = [cbuf[b, i, pl.ds(t * L, L)] for t in range(D // L)]

                @plsc.parallel_loop(0, R, unroll=3)
                def _dot(j):
                    row = i * R + j
                    p = [v[t] * ubuf[b, row, pl.ds(t * L, L)]
                         for t in range(D // L)]
                    while len(p) > 1:
                        p = [p[t] + p[t + 1] for t in range(0, len(p), 2)]
                    plsc.store_scatter(stage, [scat_base + (j * CHUNK + i)],
                                       p[0])

            s = stage[pl.ds(0, L)]
            for t in range(1, L):
                s = s + stage[pl.ds(t * u_rows, L)]
            s = jnp.minimum(jnp.maximum(s, -10.0), 10.0)
            opos[pl.ds(g * CHUNK, L)] = s

            @plsc.parallel_loop(1, R, unroll=2)
            def _reduce(q):
                s = stage[pl.ds(q * L, L)]
                for t in range(1, L):
                    s = s + stage[pl.ds(t * u_rows + q * L, L)]
                s = jnp.minimum(jnp.maximum(s, -10.0), 10.0)
                plsc.store_scatter(
                    oneg, [scat_o + (g * CHUNK * K + q - 1)], s)

        fire(0, 0)

        @pl.loop(0, n_chunks, step=2)
        def _pair(g):
            fire(g + 1, 1)
            drain(0)
            compute(g, 0)

            @pl.when(g + 2 < n_chunks)
            def _():
                fire(g + 2, 0)

            drain(1)
            compute(g + 1, 1)

        pltpu.sync_copy(opos, pos_out.at[pl.ds(sbase, spw)])
        pltpu.sync_copy(oneg, neg_out.at[pl.ds(sbase * K, spw * K)])

    cp = pltpu.CompilerParams()
    if "needs_layout_passes" in pltpu.CompilerParams.__dataclass_fields__:
        cp = dataclasses.replace(cp, needs_layout_passes=False)
    return pl.kernel(
        body,
        out_type=(jax.ShapeDtypeStruct((B,), jnp.float32),
                  jax.ShapeDtypeStruct((B * K,), jnp.float32)),
        mesh=mesh,
        compiler_params=cp,
        scratch_types=[
            pltpu.VMEM((spw,), jnp.int32),
            pltpu.VMEM((spw * R,), jnp.int32),
            pltpu.VMEM((2, CHUNK, D), jnp.float32),
            pltpu.VMEM((2, u_rows, D), jnp.float32),
            pltpu.VMEM((L * u_rows,), jnp.float32),
            pltpu.VMEM((spw,), jnp.float32),
            pltpu.VMEM((spw * K,), jnp.float32),
            pltpu.SemaphoreType.DMA,
            pltpu.SemaphoreType.DMA,
        ],
    )


def kernel(center_words, context_words, negative_samples, center_emb,
           context_emb):
    B = center_words.shape[0]
    cw = center_words.astype(jnp.int32)
    u_idx = jnp.concatenate(
        [context_words.astype(jnp.int32)[:, None],
         negative_samples.astype(jnp.int32)], axis=1).reshape(B * R)
    pos, neg = _build_sc_call(B)(cw, u_idx, center_emb, context_emb)
    return pos, neg.reshape(B, K)
